# baseline (speedup 1.0000x reference)
import functools

import jax
import jax.numpy as jnp
from jax import lax
from jax.experimental import pallas as pl
from jax.experimental.pallas import tpu as pltpu
from jax.experimental.pallas import tpu_sc as plsc

N = 10000
E = 320000
D = 128
F = 128

NC = 2
NS = 16
NW = NC * NS
EPW = E // NW
CH = 80
NCH = EPW // CH
RPT = 624
LANES = 16
DW = D // 2

_mesh = plsc.VectorSubcoreMesh(core_axis_name="c", subcore_axis_name="s")


@functools.partial(
    pl.kernel,
    out_type=jax.ShapeDtypeStruct((NC, N, D), jnp.float32),
    mesh=_mesh,
    compiler_params=pltpu.CompilerParams(use_tc_tiling_on_sc=False),
    scratch_types=[
        pltpu.VMEM((EPW,), jnp.int32),
        pltpu.VMEM((EPW,), jnp.int32),
        pltpu.VMEM((EPW,), jnp.float32),
        pltpu.VMEM((CH,), jnp.int32),
        pltpu.VMEM((CH, DW), jnp.int32),
        pltpu.VMEM((CH, DW), jnp.int32),
        pltpu.VMEM((CH, D), jnp.float32),
        pltpu.VMEM_SHARED((N, D), jnp.float32),
        pltpu.SemaphoreType.DMA,
        pltpu.SemaphoreType.DMA,
    ],
)
def _sc_aggregate(feat_hbm, col_hbm, row_hbm, w_hbm, out_hbm,
                  col_v, row_v, w_v, rowbuf, bbuf0, bbuf1, sbuf,
                  agg_sh, sem0, sem1):
    cid = lax.axis_index("c")
    sid = lax.axis_index("s")
    wid = sid * NC + cid

    pltpu.sync_copy(col_hbm.at[wid], col_v)
    pltpu.sync_copy(row_hbm.at[wid], row_v)
    pltpu.sync_copy(w_hbm.at[wid], w_v)

    zero16 = jnp.zeros((LANES,), jnp.float32)

    def zbody(i, carry):
        for j in range(D // LANES):
            sbuf[i, pl.ds(j * LANES, LANES)] = zero16
        return carry

    lax.fori_loop(0, CH, zbody, 0)
    base_rows = sid * RPT
    for k in range((RPT + CH - 1) // CH):
        sz = min(CH, RPT - k * CH)
        pltpu.sync_copy(sbuf.at[pl.ds(0, sz)],
                        agg_sh.at[pl.ds(base_rows + k * CH, sz)])
    @pl.when(sid == NS - 1)
    def _zero_tail():
        pltpu.sync_copy(sbuf.at[pl.ds(0, N - NS * RPT)],
                        agg_sh.at[pl.ds(NS * RPT, N - NS * RPT)])

    plsc.subcore_barrier()

    def issue_gather(c, buf, sem):
        pltpu.async_copy(feat_hbm.at[col_v.at[pl.ds(c * CH, CH)]], buf, sem)

    def wait_gather(buf, sem):
        pltpu.make_async_copy(feat_hbm.at[col_v.at[pl.ds(0, CH)]], buf,
                              sem).wait()

    def process(c, bbuf):
        base = c * CH
        for g in range(CH // LANES):
            rowbuf[pl.ds(g * LANES, LANES)] = (
                row_v[pl.ds(base + g * LANES, LANES)])

        def sgroup(g, inner):
            wvec = w_v[pl.ds(base + g * LANES, LANES)]
            for r in range(LANES):
                wr = jnp.full((LANES,), wvec[r], dtype=jnp.float32)
                row = g * LANES + r
                for q in range(D // (2 * LANES)):
                    vi = bbuf[row, pl.ds(q * LANES, LANES)]
                    a = lax.bitcast_convert_type(vi << 16, jnp.float32)
                    b = lax.bitcast_convert_type(vi & jnp.int32(-65536),
                                                 jnp.float32)
                    sbuf[row, pl.ds(q * 2 * LANES, LANES)] = a * wr
                    sbuf[row, pl.ds(q * 2 * LANES + LANES, LANES)] = b * wr
            return inner

        lax.fori_loop(0, CH // LANES, sgroup, 0)
        pltpu.sync_copy(sbuf, agg_sh.at[rowbuf], add=True)

    issue_gather(0, bbuf0, sem0)

    def body2(k, carry):
        c = 2 * k
        wait_gather(bbuf0, sem0)
        issue_gather(c + 1, bbuf1, sem1)
        process(c, bbuf0)
        wait_gather(bbuf1, sem1)
        issue_gather(c + 2, bbuf0, sem0)
        process(c + 1, bbuf1)
        return carry

    lax.fori_loop(0, (NCH - 1) // 2, body2, 0)
    wait_gather(bbuf0, sem0)
    process(NCH - 1, bbuf0)
    plsc.subcore_barrier()

    pltpu.sync_copy(agg_sh.at[pl.ds(base_rows, RPT)],
                    out_hbm.at[cid, pl.ds(base_rows, RPT)])
    @pl.when(sid == NS - 1)
    def _write_tail():
        pltpu.sync_copy(agg_sh.at[pl.ds(NS * RPT, N - NS * RPT)],
                        out_hbm.at[cid, pl.ds(NS * RPT, N - NS * RPT)])


def _tc_project_body(agg_ref, w_ref, b_ref, out_ref):
    x = agg_ref[0] + agg_ref[1]
    y = jnp.dot(x, w_ref[...], preferred_element_type=jnp.float32)
    out_ref[...] = jnp.maximum(y + b_ref[...], 0.0)


_TC_BLOCK = 2000


def _tc_project(partials, weights, bias2d):
    grid = N // _TC_BLOCK
    return pl.pallas_call(
        _tc_project_body,
        grid=(grid,),
        in_specs=[
            pl.BlockSpec((NC, _TC_BLOCK, D), lambda i: (0, i, 0)),
            pl.BlockSpec((D, F), lambda i: (0, 0)),
            pl.BlockSpec((1, F), lambda i: (0, 0)),
        ],
        out_specs=pl.BlockSpec((_TC_BLOCK, F), lambda i: (i, 0)),
        out_shape=jax.ShapeDtypeStruct((N, F), jnp.float32),
    )(partials, weights, bias2d)


def kernel(features, edge_index, edge_weight, kernel, bias):
    col = edge_index[1].reshape(NW, EPW)
    row = edge_index[0].reshape(NW, EPW)
    w = edge_weight.reshape(NW, EPW)
    featb = (features.astype(jnp.bfloat16)
             .reshape(N, D // 32, 2, 16).swapaxes(2, 3).reshape(N, D))
    feat32 = jax.lax.bitcast_convert_type(
        featb.reshape(N, DW, 2), jnp.int32)
    partials = _sc_aggregate(feat32, col, row, w)
    return _tc_project(partials, kernel, bias.reshape(1, F))


# speedup vs baseline: 2.1642x; 2.1642x over previous
import functools

import jax
import jax.numpy as jnp
from jax import lax
from jax.experimental import pallas as pl
from jax.experimental.pallas import tpu as pltpu
from jax.experimental.pallas import tpu_sc as plsc

N = 10000
E = 320000
D = 128
F = 128

NC = 2
NS = 16
NW = NC * NS
EPW = E // NW
CH = 80
NCH = EPW // CH
RPT = 624
LANES = 16
NBUF = 3

_mesh = plsc.VectorSubcoreMesh(core_axis_name="c", subcore_axis_name="s")


@functools.partial(
    pl.kernel,
    out_type=jax.ShapeDtypeStruct((NC, N, D), jnp.float32),
    mesh=_mesh,
    scratch_types=[
        pltpu.VMEM((EPW,), jnp.int32),
        [pltpu.VMEM((CH,), jnp.int32) for _ in range(NBUF)],
        [pltpu.VMEM((CH,), jnp.float32) for _ in range(NBUF)],
        [pltpu.VMEM((CH, D), jnp.float32) for _ in range(NBUF)],
        [pltpu.SemaphoreType.DMA for _ in range(NBUF)],
        [pltpu.SemaphoreType.DMA for _ in range(NBUF)],
        pltpu.VMEM_SHARED((N, D), jnp.float32),
    ],
)
def _sc_aggregate(feat_hbm, col_hbm, roww_hbm, w_hbm, out_hbm,
                  col_v, rowbufs, wbufs, bufs, gsems, isems, agg_sh):
    cid = lax.axis_index("c")
    sid = lax.axis_index("s")
    wid = sid * NC + cid

    pltpu.sync_copy(col_hbm.at[wid], col_v)

    zero16 = jnp.zeros((LANES,), jnp.float32)

    def zbody(i, carry):
        for j in range(D // LANES):
            bufs[0][i, pl.ds(j * LANES, LANES)] = zero16
        return carry

    lax.fori_loop(0, CH, zbody, 0)
    base_rows = sid * RPT
    for k in range((RPT + CH - 1) // CH):
        sz = min(CH, RPT - k * CH)
        pltpu.sync_copy(bufs[0].at[pl.ds(0, sz)],
                        agg_sh.at[pl.ds(base_rows + k * CH, sz)])
    @pl.when(sid == NS - 1)
    def _zero_tail():
        pltpu.sync_copy(bufs[0].at[pl.ds(0, N - NS * RPT)],
                        agg_sh.at[pl.ds(NS * RPT, N - NS * RPT)])

    plsc.subcore_barrier()

    def issue(c, j):
        pltpu.async_copy(feat_hbm.at[col_v.at[pl.ds(c * CH, CH)]],
                         bufs[j], gsems[j])
        pltpu.async_copy(roww_hbm.at[wid * NCH + c], rowbufs[j], isems[j])
        pltpu.async_copy(w_hbm.at[wid * NCH + c], wbufs[j], isems[j])

    def wait(j):
        pltpu.make_async_copy(feat_hbm.at[col_v.at[pl.ds(0, CH)]],
                              bufs[j], gsems[j]).wait()
        pltpu.make_async_copy(roww_hbm.at[0], rowbufs[j], isems[j]).wait()
        pltpu.make_async_copy(w_hbm.at[0], wbufs[j], isems[j]).wait()

    def process(c, j):
        buf = bufs[j]

        def sgroup(g, inner):
            wvec = wbufs[j][pl.ds(g * LANES, LANES)]
            for r in range(LANES):
                wr = jnp.full((LANES,), wvec[r], dtype=jnp.float32)
                row = g * LANES + r
                for q in range(D // LANES):
                    sl = pl.ds(q * LANES, LANES)
                    buf[row, sl] = buf[row, sl] * wr
            return inner

        lax.fori_loop(0, CH // LANES, sgroup, 0)
        pltpu.sync_copy(buf, agg_sh.at[rowbufs[j]], add=True)

    issue(0, 0)
    issue(1, 1)

    def body3(k, carry):
        for j in range(NBUF):
            c = NBUF * k + j
            wait(j)
            issue(c + 2, (j + 2) % NBUF)
            process(c, j)
        return carry

    NLOOP = (NCH - 2) // NBUF
    lax.fori_loop(0, NLOOP, body3, 0)
    for c in range(NBUF * NLOOP, NCH):
        j = c % NBUF
        wait(j)
        process(c, j)
    plsc.subcore_barrier()

    pltpu.sync_copy(agg_sh.at[pl.ds(base_rows, RPT)],
                    out_hbm.at[cid, pl.ds(base_rows, RPT)])
    @pl.when(sid == NS - 1)
    def _write_tail():
        pltpu.sync_copy(agg_sh.at[pl.ds(NS * RPT, N - NS * RPT)],
                        out_hbm.at[cid, pl.ds(NS * RPT, N - NS * RPT)])


def _tc_project_body(agg_ref, w_ref, b_ref, out_ref):
    x = agg_ref[0] + agg_ref[1]
    y = jnp.dot(x, w_ref[...], preferred_element_type=jnp.float32)
    out_ref[...] = jnp.maximum(y + b_ref[...], 0.0)


_TC_BLOCK = 2000


def _tc_project(partials, weights, bias2d):
    grid = N // _TC_BLOCK
    return pl.pallas_call(
        _tc_project_body,
        grid=(grid,),
        in_specs=[
            pl.BlockSpec((NC, _TC_BLOCK, D), lambda i: (0, i, 0)),
            pl.BlockSpec((D, F), lambda i: (0, 0)),
            pl.BlockSpec((1, F), lambda i: (0, 0)),
        ],
        out_specs=pl.BlockSpec((_TC_BLOCK, F), lambda i: (i, 0)),
        out_shape=jax.ShapeDtypeStruct((N, F), jnp.float32),
    )(partials, weights, bias2d)


def kernel(features, edge_index, edge_weight, kernel, bias):
    col = edge_index[1].reshape(NW, EPW)
    roww = edge_index[0].reshape(NW * NCH, CH)
    w = edge_weight.reshape(NW * NCH, CH)
    partials = _sc_aggregate(features, col, roww, w)
    return _tc_project(partials, kernel, bias.reshape(1, F))
